# Initial kernel scaffold; baseline (speedup 1.0000x reference)
import jax
import jax.numpy as jnp
from jax.experimental import pallas as pl


def kernel(x, edge_index, edge_attr, ptr, W1, b1, W2, b2, S1w, S1b, S2w, S2b, S3w, S3b, M1w, M1b, M2w, M2b, M3w, M3b):
    raise NotImplementedError("write your pallas kernel here")


# speedup vs baseline: 12.0730x; 12.0730x over previous
import functools

import jax
import jax.numpy as jnp
from jax import lax
from jax.experimental import pallas as pl
from jax.experimental.pallas import tpu as pltpu
from jax.experimental.pallas import tpu_sc as plsc

N = 10000
E = 320000
D = 128
NG = 8
SEG = N // NG

NC = 2
NS = 16
NW = NC * NS
CHUNK = 128
NCHUNKS = E // CHUNK
NPAD = 10240

_mesh = plsc.VectorSubcoreMesh(core_axis_name="c", subcore_axis_name="s")


@functools.partial(
    pl.kernel,
    out_type=jax.ShapeDtypeStruct((NC, NPAD), jnp.float32),
    mesh=_mesh,
    scratch_types=[
        pltpu.VMEM((CHUNK,), jnp.int32),
        pltpu.VMEM((CHUNK,), jnp.float32),
        pltpu.VMEM((640,), jnp.float32),
        pltpu.VMEM_SHARED((NPAD,), jnp.float32),
    ],
)
def _sc_deg(dst_hbm, ew_hbm, out_hbm, didx, ewv, dbuf, acc):
    cid = lax.axis_index("c")
    sid = lax.axis_index("s")
    w = sid * NC + cid

    def _zero(i, carry):
        dbuf[pl.ds(i * 16, 16)] = jnp.zeros((16,), jnp.float32)
        return carry

    lax.fori_loop(0, 40, _zero, 0)
    pltpu.sync_copy(dbuf, acc.at[pl.ds(sid * 640, 640)])
    plsc.subcore_barrier()

    nch = (NCHUNKS - w + NW - 1) // NW

    def _chunk(i, carry):
        base = (w + i * NW) * CHUNK
        pltpu.sync_copy(dst_hbm.at[pl.ds(base, CHUNK)], didx)
        pltpu.sync_copy(ew_hbm.at[pl.ds(base, CHUNK)], ewv)
        pltpu.sync_copy(ewv, acc.at[didx], add=True)
        return carry

    lax.fori_loop(0, nch, _chunk, 0)
    plsc.subcore_barrier()

    pltpu.sync_copy(acc.at[pl.ds(sid * 640, 640)], dbuf)
    pltpu.sync_copy(dbuf, out_hbm.at[cid, pl.ds(sid * 640, 640)])


@functools.partial(
    pl.kernel,
    out_type=jax.ShapeDtypeStruct((NC, N, D), jnp.float32),
    mesh=_mesh,
    scratch_types=[
        pltpu.VMEM((CHUNK,), jnp.int32),
        pltpu.VMEM((CHUNK,), jnp.int32),
        pltpu.VMEM((CHUNK,), jnp.float32),
        pltpu.VMEM((CHUNK, D), jnp.float32),
        pltpu.VMEM_SHARED((NPAD, D), jnp.float32),
        pltpu.SemaphoreType.DMA,
    ],
)
def _sc_edge(hw_hbm, src_hbm, dst_hbm, ew_hbm, out_hbm,
             sidx, didx, ewv, rows, acc, sem):
    cid = lax.axis_index("c")
    sid = lax.axis_index("s")
    w = sid * NC + cid

    def _zrow(j, carry):
        for cc in range(D // 16):
            rows[j, pl.ds(cc * 16, 16)] = jnp.zeros((16,), jnp.float32)
        return carry

    lax.fori_loop(0, CHUNK, _zrow, 0)
    for k in range(5):
        pltpu.sync_copy(rows, acc.at[pl.ds(sid * 640 + k * CHUNK, CHUNK)])
    plsc.subcore_barrier()

    nch = (NCHUNKS - w + NW - 1) // NW

    def _chunk(i, carry):
        base = (w + i * NW) * CHUNK
        pltpu.sync_copy(src_hbm.at[pl.ds(base, CHUNK)], sidx)
        pltpu.sync_copy(dst_hbm.at[pl.ds(base, CHUNK)], didx)
        pltpu.sync_copy(ew_hbm.at[pl.ds(base, CHUNK)], ewv)
        pltpu.async_copy(hw_hbm.at[sidx], rows, sem).wait()

        def _scale(jj, c2):
            wv = ewv[pl.ds(jj * 16, 16)]
            for l in range(16):
                s = wv[l]
                j = jj * 16 + l
                for cc in range(D // 16):
                    rows[j, pl.ds(cc * 16, 16)] = rows[j, pl.ds(cc * 16, 16)] * s
            return c2

        lax.fori_loop(0, CHUNK // 16, _scale, 0)
        pltpu.sync_copy(rows, acc.at[didx], add=True)
        return carry

    lax.fori_loop(0, nch, _chunk, 0)
    plsc.subcore_barrier()

    @pl.when(sid < 15)
    def _dump_a():
        for k in range(6):
            r = sid * 624 + k * 104
            pltpu.sync_copy(acc.at[pl.ds(r, 104)], rows.at[pl.ds(0, 104)])
            pltpu.sync_copy(rows.at[pl.ds(0, 104)],
                            out_hbm.at[cid, pl.ds(r, 104)])

    @pl.when(sid == 15)
    def _dump_b():
        for k in range(5):
            r = 15 * 624 + k * 128
            pltpu.sync_copy(acc.at[pl.ds(r, 128)], rows)
            pltpu.sync_copy(rows, out_hbm.at[cid, pl.ds(r, 128)])


def _tc_a_body(degp_ref, x_ref, w1_ref, hw_ref, dinv_ref):
    deg = degp_ref[0, :N] + degp_ref[1, :N] + 1.0
    dinv = jnp.where(deg > 0, lax.rsqrt(deg), 0.0)
    hw = jnp.dot(x_ref[...], w1_ref[...], preferred_element_type=jnp.float32)
    hw_ref[...] = hw * dinv[:, None]
    dinv_ref[...] = dinv[:, None]


def _tc_a(degp, x, W1):
    return pl.pallas_call(
        _tc_a_body,
        out_shape=[
            jax.ShapeDtypeStruct((N, D), jnp.float32),
            jax.ShapeDtypeStruct((N, 1), jnp.float32),
        ],
    )(degp, x, W1)


def _tc_b_body(ap_ref, hw_ref, dinv_ref, b_ref, w2_ref, out_ref):
    s = ap_ref[0] + ap_ref[1] + hw_ref[...]
    h1 = jnp.maximum(dinv_ref[...] * s + b_ref[...], 0.0)
    hw2 = jnp.dot(h1, w2_ref[...], preferred_element_type=jnp.float32)
    out_ref[...] = dinv_ref[...] * hw2


def _tc_b(ap, hw, dinv, b1, W2):
    blk = 1000
    grid = N // blk
    return pl.pallas_call(
        _tc_b_body,
        grid=(grid,),
        in_specs=[
            pl.BlockSpec((NC, blk, D), lambda i: (0, i, 0)),
            pl.BlockSpec((blk, D), lambda i: (i, 0)),
            pl.BlockSpec((blk, 1), lambda i: (i, 0)),
            pl.BlockSpec((D,), lambda i: (0,)),
            pl.BlockSpec((D, D), lambda i: (0, 0)),
        ],
        out_specs=pl.BlockSpec((blk, D), lambda i: (i, 0)),
        out_shape=jax.ShapeDtypeStruct((N, D), jnp.float32),
    )(ap, hw, dinv, b1, W2)


def _tc_c_body(ap_ref, hw_ref, dinv_ref, b_ref,
               s1w_ref, s1b_ref, s2w_ref, s2b_ref, s3w_ref, s3b_ref,
               hh_ref):
    s = ap_ref[0, 0] + ap_ref[1, 0] + hw_ref[0]
    h2 = jnp.maximum(dinv_ref[0] * s + b_ref[...], 0.0)
    nrm = jnp.sqrt(jnp.sum(h2 * h2, axis=1, keepdims=True))
    hn = h2 / jnp.maximum(nrm, 1e-12)
    g = jnp.maximum(jnp.dot(hn, s1w_ref[...],
                            preferred_element_type=jnp.float32) + s1b_ref[...], 0.0)
    g = jnp.maximum(jnp.dot(g, s2w_ref[...],
                            preferred_element_type=jnp.float32) + s2b_ref[...], 0.0)
    g = jnp.maximum(jnp.dot(g, s3w_ref[...],
                            preferred_element_type=jnp.float32) + s3b_ref[...], 0.0)
    hh = lax.dot_general(g, g, (((0,), (0,)), ((), ())),
                         preferred_element_type=jnp.float32)
    hh_ref[...] = hh[None]


def _tc_c(ap, hw, dinv, b2, S1w, S1b, S2w, S2b, S3w, S3b):
    ap3 = ap.reshape(NC, NG, SEG, D)
    hw3 = hw.reshape(NG, SEG, D)
    dinv3 = dinv.reshape(NG, SEG, 1)
    return pl.pallas_call(
        _tc_c_body,
        grid=(NG,),
        in_specs=[
            pl.BlockSpec((NC, 1, SEG, D), lambda i: (0, i, 0, 0)),
            pl.BlockSpec((1, SEG, D), lambda i: (i, 0, 0)),
            pl.BlockSpec((1, SEG, 1), lambda i: (i, 0, 0)),
            pl.BlockSpec((D,), lambda i: (0,)),
            pl.BlockSpec((D, 32), lambda i: (0, 0)),
            pl.BlockSpec((32,), lambda i: (0,)),
            pl.BlockSpec((32, 32), lambda i: (0, 0)),
            pl.BlockSpec((32,), lambda i: (0,)),
            pl.BlockSpec((32, 32), lambda i: (0, 0)),
            pl.BlockSpec((32,), lambda i: (0,)),
        ],
        out_specs=pl.BlockSpec((1, 32, 32), lambda i: (i, 0, 0)),
        out_shape=jax.ShapeDtypeStruct((NG, 32, 32), jnp.float32),
    )(ap3, hw3, dinv3, b2, S1w, S1b, S2w, S2b, S3w, S3b)


def _tc_d_body(hh_ref, m1w_ref, m1b_ref, m2w_ref, m2b_ref, m3w_ref, m3b_ref,
               out_ref):
    o = jnp.maximum(jnp.dot(hh_ref[...], m1w_ref[...],
                            preferred_element_type=jnp.float32) + m1b_ref[...], 0.0)
    o = jnp.maximum(jnp.dot(o, m2w_ref[...],
                            preferred_element_type=jnp.float32) + m2b_ref[...], 0.0)
    o = jnp.maximum(jnp.dot(o, m3w_ref[...],
                            preferred_element_type=jnp.float32) + m3b_ref[...], 0.0)
    out_ref[...] = o


def _tc_d(HH, M1w, M1b, M2w, M2b, M3w, M3b):
    return pl.pallas_call(
        _tc_d_body,
        out_shape=jax.ShapeDtypeStruct((NG, 2), jnp.float32),
    )(HH, M1w, M1b, M2w, M2b, M3w, M3b)


@jax.jit
def kernel(x, edge_index, edge_attr, ptr, W1, b1, W2, b2,
           S1w, S1b, S2w, S2b, S3w, S3b, M1w, M1b, M2w, M2b, M3w, M3b):
    src = edge_index[0]
    dst = edge_index[1]

    degp = _sc_deg(dst, edge_attr)
    hw1, dinv = _tc_a(degp, x, W1)
    A1 = _sc_edge(hw1, src, dst, edge_attr)
    hw2 = _tc_b(A1, hw1, dinv, b1, W2)
    A2 = _sc_edge(hw2, src, dst, edge_attr)
    HH3 = _tc_c(A2, hw2, dinv, b2, S1w, S1b, S2w, S2b, S3w, S3b)
    HH = HH3.reshape(NG, 32 * 32)
    out = _tc_d(HH, M1w, M1b, M2w, M2b, M3w, M3b)
    return (HH, out)


# speedup vs baseline: 27.4351x; 2.2724x over previous
import functools

import jax
import jax.numpy as jnp
from jax import lax
from jax.experimental import pallas as pl
from jax.experimental.pallas import tpu as pltpu
from jax.experimental.pallas import tpu_sc as plsc

N = 10000
E = 320000
D = 128
NG = 8
SEG = N // NG

NC = 2
NS = 16
NW = NC * NS
CHUNK = 128
NCHUNKS = E // CHUNK
NPAD = 10240

_mesh = plsc.VectorSubcoreMesh(core_axis_name="c", subcore_axis_name="s")


@functools.partial(
    pl.kernel,
    out_type=jax.ShapeDtypeStruct((NC, NPAD), jnp.float32),
    mesh=_mesh,
    scratch_types=[
        pltpu.VMEM((3, CHUNK), jnp.int32),
        pltpu.VMEM((3, CHUNK), jnp.float32),
        pltpu.VMEM((640,), jnp.float32),
        pltpu.VMEM_SHARED((NPAD,), jnp.float32),
        pltpu.SemaphoreType.DMA,
        pltpu.SemaphoreType.DMA,
        pltpu.SemaphoreType.DMA,
        pltpu.SemaphoreType.DMA,
        pltpu.SemaphoreType.DMA,
        pltpu.SemaphoreType.DMA,
        pltpu.SemaphoreType.DMA,
        pltpu.SemaphoreType.DMA,
        pltpu.SemaphoreType.DMA,
    ],
)
def _sc_deg(dst_hbm, ew_hbm, out_hbm, didx, ewb, dbuf, acc,
            sia0, sia1, sia2, sib0, sib1, sib2, ss0, ss1, ss2):
    sia = (sia0, sia1, sia2)
    sib = (sib0, sib1, sib2)
    ss = (ss0, ss1, ss2)
    cid = lax.axis_index("c")
    sid = lax.axis_index("s")
    w = sid * NC + cid

    def _zero(i, carry):
        dbuf[pl.ds(i * 16, 16)] = jnp.zeros((16,), jnp.float32)
        return carry

    lax.fori_loop(0, 40, _zero, 0)
    pltpu.sync_copy(dbuf, acc.at[pl.ds(sid * 640, 640)])
    plsc.subcore_barrier()

    nch = (NCHUNKS - w + NW - 1) // NW

    def _idx_copy(j, slot):
        base = (w + j * NW) * CHUNK
        return pltpu.make_async_copy(dst_hbm.at[pl.ds(base, CHUNK)],
                                     didx.at[slot], sia[slot])

    def _ew_copy(j, slot):
        base = (w + j * NW) * CHUNK
        return pltpu.make_async_copy(ew_hbm.at[pl.ds(base, CHUNK)],
                                     ewb.at[slot], sib[slot])

    def _scat_copy(slot):
        return pltpu.async_copy(ewb.at[slot], acc.at[didx.at[slot]],
                                ss[slot], add=True)

    def _scat_wait(slot):
        pltpu.make_async_copy(ewb.at[slot], acc.at[didx.at[slot]],
                              ss[slot]).wait()

    _idx_copy(0, 0).start()
    _ew_copy(0, 0).start()

    def _step(s, u, carry):
        j = s * 3 + u
        u1 = (u + 1) % 3

        @pl.when((j >= 2) & (j + 1 < nch))
        def _():
            _scat_wait(u1)

        @pl.when(j + 1 < nch)
        def _():
            _idx_copy(j + 1, u1).start()
            _ew_copy(j + 1, u1).start()

        @pl.when(j < nch)
        def _():
            _idx_copy(j, u).wait()
            _ew_copy(j, u).wait()
            _scat_copy(u)

        return carry

    def _steps3(s, carry):
        for u in range(3):
            _step(s, u, carry)
        return carry

    lax.fori_loop(0, 27, _steps3, 0)

    for u in range(3):
        _scat_wait(u)
    plsc.subcore_barrier()

    pltpu.sync_copy(acc.at[pl.ds(sid * 640, 640)], dbuf)
    pltpu.sync_copy(dbuf, out_hbm.at[cid, pl.ds(sid * 640, 640)])


@functools.partial(
    pl.kernel,
    out_type=jax.ShapeDtypeStruct((NC, N, D), jnp.float32),
    mesh=_mesh,
    scratch_types=[
        pltpu.VMEM((3, 2, CHUNK), jnp.int32),
        pltpu.VMEM((3, CHUNK), jnp.float32),
        pltpu.VMEM((3, CHUNK, D), jnp.float32),
        pltpu.VMEM_SHARED((N, D), jnp.float32),
        pltpu.SemaphoreType.DMA,
        pltpu.SemaphoreType.DMA,
        pltpu.SemaphoreType.DMA,
        pltpu.SemaphoreType.DMA,
        pltpu.SemaphoreType.DMA,
        pltpu.SemaphoreType.DMA,
        pltpu.SemaphoreType.DMA,
        pltpu.SemaphoreType.DMA,
        pltpu.SemaphoreType.DMA,
        pltpu.SemaphoreType.DMA,
        pltpu.SemaphoreType.DMA,
        pltpu.SemaphoreType.DMA,
    ],
)
def _sc_edge(hw_hbm, pk_hbm, ew_hbm, out_hbm, pk, ewb, rows, acc,
             sia0, sia1, sia2, sib0, sib1, sib2, sg0, sg1, sg2,
             ss0, ss1, ss2):
    sia = (sia0, sia1, sia2)
    sib = (sib0, sib1, sib2)
    sg = (sg0, sg1, sg2)
    ss = (ss0, ss1, ss2)
    cid = lax.axis_index("c")
    sid = lax.axis_index("s")
    w = sid * NC + cid

    def _zrow(j, carry):
        for cc in range(D // 16):
            rows[0, j, pl.ds(cc * 16, 16)] = jnp.zeros((16,), jnp.float32)
        return carry

    lax.fori_loop(0, CHUNK, _zrow, 0)

    @pl.when(sid < 15)
    def _zero_a():
        for k in range(6):
            pltpu.sync_copy(rows.at[0, pl.ds(0, 104)],
                            acc.at[pl.ds(sid * 624 + k * 104, 104)])

    @pl.when(sid == 15)
    def _zero_b():
        for k in range(5):
            pltpu.sync_copy(rows.at[0], acc.at[pl.ds(15 * 624 + k * 128, 128)])

    plsc.subcore_barrier()

    nch = (NCHUNKS - w + NW - 1) // NW

    def _idx_copy(j, slot):
        base = (w + j * NW) * CHUNK
        return pltpu.make_async_copy(pk_hbm.at[:, pl.ds(base, CHUNK)],
                                     pk.at[slot], sia[slot])

    def _ew_copy(j, slot):
        base = (w + j * NW) * CHUNK
        return pltpu.make_async_copy(ew_hbm.at[pl.ds(base, CHUNK)],
                                     ewb.at[slot], sib[slot])

    def _gath_copy(slot):
        return pltpu.make_async_copy(hw_hbm.at[pk.at[slot, 0]],
                                     rows.at[slot], sg[slot])

    def _scat_copy(slot):
        return pltpu.async_copy(rows.at[slot], acc.at[pk.at[slot, 1]],
                                ss[slot], add=True)

    def _scat_wait(slot):
        pltpu.make_async_copy(rows.at[slot], acc.at[pk.at[slot, 1]],
                              ss[slot]).wait()

    _idx_copy(0, 0).start()
    _ew_copy(0, 0).start()

    def _step(s, u, carry):
        j = s * 3 + u
        u1 = (u + 1) % 3
        u2 = (u + 2) % 3

        @pl.when((j >= 2) & (j + 1 < nch))
        def _():
            _scat_wait(u1)

        @pl.when(j + 1 < nch)
        def _():
            _idx_copy(j + 1, u1).start()
            _ew_copy(j + 1, u1).start()

        @pl.when(j < nch)
        def _():
            _idx_copy(j, u).wait()
            _ew_copy(j, u).wait()
            _gath_copy(u).start()

        @pl.when((j >= 1) & (j - 1 < nch))
        def _():
            _gath_copy(u2).wait()

            def _scale(jj, c2):
                wv = ewb[u2, pl.ds(jj * 16, 16)]
                for l in range(16):
                    s_ = wv[l]
                    jr = jj * 16 + l
                    for cc in range(D // 16):
                        rows[u2, jr, pl.ds(cc * 16, 16)] = (
                            rows[u2, jr, pl.ds(cc * 16, 16)] * s_)
                return c2

            lax.fori_loop(0, CHUNK // 16, _scale, 0)
            _scat_copy(u2)

        return carry

    def _steps3(s, carry):
        for u in range(3):
            _step(s, u, carry)
        return carry

    lax.fori_loop(0, 27, _steps3, 0)

    for u in range(3):
        _scat_wait(u)

    plsc.subcore_barrier()

    @pl.when(sid < 15)
    def _dump_a():
        for k in range(6):
            r = sid * 624 + k * 104
            pltpu.sync_copy(acc.at[pl.ds(r, 104)], rows.at[0, pl.ds(0, 104)])
            pltpu.sync_copy(rows.at[0, pl.ds(0, 104)],
                            out_hbm.at[cid, pl.ds(r, 104)])

    @pl.when(sid == 15)
    def _dump_b():
        for k in range(5):
            r = 15 * 624 + k * 128
            pltpu.sync_copy(acc.at[pl.ds(r, 128)], rows.at[0])
            pltpu.sync_copy(rows.at[0], out_hbm.at[cid, pl.ds(r, 128)])


def _tc_a_body(degp_ref, x_ref, w1_ref, hw_ref, dinv_ref):
    deg = degp_ref[0, :N] + degp_ref[1, :N] + 1.0
    dinv = jnp.where(deg > 0, lax.rsqrt(deg), 0.0)
    hw = jnp.dot(x_ref[...], w1_ref[...], preferred_element_type=jnp.float32)
    hw_ref[...] = hw * dinv[:, None]
    dinv_ref[...] = dinv[:, None]


def _tc_a(degp, x, W1):
    return pl.pallas_call(
        _tc_a_body,
        out_shape=[
            jax.ShapeDtypeStruct((N, D), jnp.float32),
            jax.ShapeDtypeStruct((N, 1), jnp.float32),
        ],
    )(degp, x, W1)


def _tc_b_body(ap_ref, hw_ref, dinv_ref, b_ref, w2_ref, out_ref):
    s = ap_ref[0] + ap_ref[1] + hw_ref[...]
    h1 = jnp.maximum(dinv_ref[...] * s + b_ref[...], 0.0)
    hw2 = jnp.dot(h1, w2_ref[...], preferred_element_type=jnp.float32)
    out_ref[...] = dinv_ref[...] * hw2


def _tc_b(ap, hw, dinv, b1, W2):
    blk = 1000
    grid = N // blk
    return pl.pallas_call(
        _tc_b_body,
        grid=(grid,),
        in_specs=[
            pl.BlockSpec((NC, blk, D), lambda i: (0, i, 0)),
            pl.BlockSpec((blk, D), lambda i: (i, 0)),
            pl.BlockSpec((blk, 1), lambda i: (i, 0)),
            pl.BlockSpec((D,), lambda i: (0,)),
            pl.BlockSpec((D, D), lambda i: (0, 0)),
        ],
        out_specs=pl.BlockSpec((blk, D), lambda i: (i, 0)),
        out_shape=jax.ShapeDtypeStruct((N, D), jnp.float32),
    )(ap, hw, dinv, b1, W2)


def _tc_c_body(ap_ref, hw_ref, dinv_ref, b_ref,
               s1w_ref, s1b_ref, s2w_ref, s2b_ref, s3w_ref, s3b_ref,
               hh_ref):
    s = ap_ref[0, 0] + ap_ref[1, 0] + hw_ref[0]
    h2 = jnp.maximum(dinv_ref[0] * s + b_ref[...], 0.0)
    nrm = jnp.sqrt(jnp.sum(h2 * h2, axis=1, keepdims=True))
    hn = h2 / jnp.maximum(nrm, 1e-12)
    g = jnp.maximum(jnp.dot(hn, s1w_ref[...],
                            preferred_element_type=jnp.float32) + s1b_ref[...], 0.0)
    g = jnp.maximum(jnp.dot(g, s2w_ref[...],
                            preferred_element_type=jnp.float32) + s2b_ref[...], 0.0)
    g = jnp.maximum(jnp.dot(g, s3w_ref[...],
                            preferred_element_type=jnp.float32) + s3b_ref[...], 0.0)
    hh = lax.dot_general(g, g, (((0,), (0,)), ((), ())),
                         preferred_element_type=jnp.float32)
    hh_ref[...] = hh[None]


def _tc_c(ap, hw, dinv, b2, S1w, S1b, S2w, S2b, S3w, S3b):
    ap3 = ap.reshape(NC, NG, SEG, D)
    hw3 = hw.reshape(NG, SEG, D)
    dinv3 = dinv.reshape(NG, SEG, 1)
    return pl.pallas_call(
        _tc_c_body,
        grid=(NG,),
        in_specs=[
            pl.BlockSpec((NC, 1, SEG, D), lambda i: (0, i, 0, 0)),
            pl.BlockSpec((1, SEG, D), lambda i: (i, 0, 0)),
            pl.BlockSpec((1, SEG, 1), lambda i: (i, 0, 0)),
            pl.BlockSpec((D,), lambda i: (0,)),
            pl.BlockSpec((D, 32), lambda i: (0, 0)),
            pl.BlockSpec((32,), lambda i: (0,)),
            pl.BlockSpec((32, 32), lambda i: (0, 0)),
            pl.BlockSpec((32,), lambda i: (0,)),
            pl.BlockSpec((32, 32), lambda i: (0, 0)),
            pl.BlockSpec((32,), lambda i: (0,)),
        ],
        out_specs=pl.BlockSpec((1, 32, 32), lambda i: (i, 0, 0)),
        out_shape=jax.ShapeDtypeStruct((NG, 32, 32), jnp.float32),
    )(ap3, hw3, dinv3, b2, S1w, S1b, S2w, S2b, S3w, S3b)


def _tc_d_body(hh_ref, m1w_ref, m1b_ref, m2w_ref, m2b_ref, m3w_ref, m3b_ref,
               out_ref):
    o = jnp.maximum(jnp.dot(hh_ref[...], m1w_ref[...],
                            preferred_element_type=jnp.float32) + m1b_ref[...], 0.0)
    o = jnp.maximum(jnp.dot(o, m2w_ref[...],
                            preferred_element_type=jnp.float32) + m2b_ref[...], 0.0)
    o = jnp.maximum(jnp.dot(o, m3w_ref[...],
                            preferred_element_type=jnp.float32) + m3b_ref[...], 0.0)
    out_ref[...] = o


def _tc_d(HH, M1w, M1b, M2w, M2b, M3w, M3b):
    return pl.pallas_call(
        _tc_d_body,
        out_shape=jax.ShapeDtypeStruct((NG, 2), jnp.float32),
    )(HH, M1w, M1b, M2w, M2b, M3w, M3b)


@jax.jit
def kernel(x, edge_index, edge_attr, ptr, W1, b1, W2, b2,
           S1w, S1b, S2w, S2b, S3w, S3b, M1w, M1b, M2w, M2b, M3w, M3b):
    dst = edge_index[1]

    degp = _sc_deg(dst, edge_attr)
    hw1, dinv = _tc_a(degp, x, W1)
    A1 = _sc_edge(hw1, edge_index, edge_attr)
    hw2 = _tc_b(A1, hw1, dinv, b1, W2)
    A2 = _sc_edge(hw2, edge_index, edge_attr)
    HH3 = _tc_c(A2, hw2, dinv, b2, S1w, S1b, S2w, S2b, S3w, S3b)
    HH = HH3.reshape(NG, 32 * 32)
    out = _tc_d(HH, M1w, M1b, M2w, M2b, M3w, M3b)
    return (HH, out)
